# baseline (speedup 1.0000x reference)
import functools

import jax
import jax.numpy as jnp
from jax import lax
from jax.experimental import pallas as pl
from jax.experimental.pallas import tpu as pltpu
from jax.experimental.pallas import tpu_sc as plsc

N = 10000
E = 320000
D = 128

NC = 2
NS = 16
CHUNK = 96
CPT = 108
EPAD = NC * NS * CPT * CHUNK
NBUF = 4
NRCHUNK = N // CHUNK
TAILR = N - NRCHUNK * CHUNK


def _mm_body(x_ref, wt_ref, b_ref, o_ref):
    o_ref[...] = (
        jnp.dot(x_ref[...], wt_ref[...], preferred_element_type=jnp.float32)
        + b_ref[...]
    )


def _support(x, wt, b2):
    return pl.pallas_call(
        _mm_body,
        grid=(10,),
        in_specs=[
            pl.BlockSpec((N // 10, D), lambda i: (i, 0)),
            pl.BlockSpec((D, D), lambda i: (0, 0)),
            pl.BlockSpec((1, D), lambda i: (0, 0)),
        ],
        out_specs=pl.BlockSpec((N // 10, D), lambda i: (i, 0)),
        out_shape=jax.ShapeDtypeStruct((N, D), jnp.float32),
    )(x, wt, b2)


def _add_body(p_ref, o_ref):
    o_ref[...] = p_ref[0] + p_ref[1]


def _reduce(p):
    return pl.pallas_call(
        _add_body,
        grid=(10,),
        in_specs=[pl.BlockSpec((2, N // 10, D), lambda i: (0, i, 0))],
        out_specs=pl.BlockSpec((N // 10, D), lambda i: (i, 0)),
        out_shape=jax.ShapeDtypeStruct((N, D), jnp.float32),
    )(p)


def _bcast_lane(v, i):
    idx = jnp.full((16, 1), i, jnp.int32)
    dn = lax.GatherDimensionNumbers(
        offset_dims=(), collapsed_slice_dims=(0,), start_index_map=(0,)
    )
    return lax.gather(v, idx, dn, (1,),
                      mode=lax.GatherScatterMode.PROMISE_IN_BOUNDS)


_mesh = plsc.VectorSubcoreMesh(core_axis_name="c", subcore_axis_name="s")


@functools.partial(
    pl.kernel,
    out_type=jax.ShapeDtypeStruct((2, N, D), jnp.float32),
    mesh=_mesh,
    compiler_params=pltpu.CompilerParams(use_tc_tiling_on_sc=False),
    scratch_types=[
        pltpu.VMEM((NBUF, 3, CHUNK), jnp.int32),
        pltpu.VMEM((NBUF, CHUNK, D), jnp.float32),
        pltpu.VMEM_SHARED((N, D), jnp.float32),
        pltpu.SemaphoreType.DMA,
        pltpu.SemaphoreType.DMA,
        pltpu.SemaphoreType.DMA,
        pltpu.SemaphoreType.DMA,
        pltpu.SemaphoreType.DMA,
        pltpu.SemaphoreType.DMA,
        pltpu.SemaphoreType.DMA,
        pltpu.SemaphoreType.DMA,
        pltpu.SemaphoreType.DMA,
        pltpu.SemaphoreType.DMA,
        pltpu.SemaphoreType.DMA,
        pltpu.SemaphoreType.DMA,
    ],
)
def _spmm(sup_hbm, meta_hbm, out_hbm,
          meta_v, bufs, acc_sh,
          si0, si1, si2, si3, sg0, sg1, sg2, sg3, ss0, ss1, ss2, ss3):
    c = lax.axis_index("c")
    s = lax.axis_index("s")
    sem_i = (si0, si1, si2, si3)
    sem_g = (sg0, sg1, sg2, sg3)
    sem_s = (ss0, ss1, ss2, ss3)
    tbase = s * CPT

    zero16 = jnp.zeros((16,), jnp.float32)

    def _zrow(r, carry):
        for f in range(D // 16):
            bufs[0, r, pl.ds(f * 16, 16)] = zero16
        return carry

    lax.fori_loop(0, CHUNK, _zrow, 0)

    nblk_mine = NRCHUNK // NS + jnp.where(s < NRCHUNK - (NRCHUNK // NS) * NS,
                                          1, 0)

    def _zchunk(k, carry):
        off = (s + k * NS) * CHUNK
        pltpu.sync_copy(bufs.at[0], acc_sh.at[pl.ds(off, CHUNK)])
        return carry

    lax.fori_loop(0, nblk_mine, _zchunk, 0)

    @pl.when(s == 0)
    def _():
        pltpu.sync_copy(bufs.at[0, pl.ds(0, TAILR)],
                        acc_sh.at[pl.ds(NRCHUNK * CHUNK, TAILR)])

    def _meta_start(k, t):
        pltpu.async_copy(meta_hbm.at[c, tbase + k], meta_v.at[t], sem_i[t])

    def _meta_wait(k, t):
        pltpu.make_async_copy(meta_hbm.at[c, tbase + k], meta_v.at[t],
                              sem_i[t]).wait()

    def _gather_start(t):
        pltpu.async_copy(sup_hbm.at[meta_v.at[t, 0]], bufs.at[t], sem_g[t])

    def _gather_wait(t):
        pltpu.make_async_copy(sup_hbm.at[meta_v.at[t, 0]], bufs.at[t],
                              sem_g[t]).wait()

    def _scatter_start(t):
        pltpu.async_copy(bufs.at[t], acc_sh.at[meta_v.at[t, 1]], sem_s[t],
                         add=True)

    def _scatter_wait(t):
        pltpu.make_async_copy(bufs.at[t], acc_sh.at[meta_v.at[t, 1]],
                              sem_s[t]).wait()

    def _scale(t):
        def _grp(g, inner):
            iv = meta_v[t, 2, pl.ds(g * 16, 16)]
            vv = lax.bitcast_convert_type(iv, jnp.float32)
            for i in range(16):
                bc = _bcast_lane(vv, i)
                r = g * 16 + i
                for f in range(D // 16):
                    bufs[t, r, pl.ds(f * 16, 16)] = (
                        bufs[t, r, pl.ds(f * 16, 16)] * bc
                    )
            return inner

        lax.fori_loop(0, CHUNK // 16, _grp, 0)

    _meta_start(0, 0)
    _meta_start(1, 1)
    _meta_start(2, 2)
    _meta_wait(0, 0)
    _gather_start(0)
    _meta_wait(1, 1)
    _gather_start(1)

    plsc.subcore_barrier()

    _gather_wait(0)
    _scale(0)
    _scatter_start(0)
    _meta_wait(2, 2)
    _gather_start(2)
    _meta_start(3, 3)

    _gather_wait(1)
    _scale(1)
    _scatter_start(1)
    _meta_wait(3, 3)
    _gather_start(3)
    _scatter_wait(0)
    _meta_start(4, 0)

    _gather_wait(2)
    _scale(2)
    _scatter_start(2)
    _meta_wait(4, 0)
    _gather_start(0)
    _scatter_wait(1)
    _meta_start(5, 1)

    _gather_wait(3)
    _scale(3)
    _scatter_start(3)
    _meta_wait(5, 1)
    _gather_start(1)
    _scatter_wait(2)
    _meta_start(6, 2)

    def _outer(j, carry):
        for t in range(NBUF):
            k = NBUF * j + t
            w = (t + 2) % NBUF
            v = (t + 3) % NBUF
            _gather_wait(t)
            _scale(t)
            _scatter_start(t)
            _meta_wait(k + 2, w)
            _gather_start(w)
            _scatter_wait(v)
            _meta_start(k + 3, v)
        return carry

    lax.fori_loop(1, CPT // NBUF - 1, _outer, 0)

    _gather_wait(0)
    _scale(0)
    _scatter_start(0)
    _meta_wait(CPT - 2, 2)
    _gather_start(2)
    _scatter_wait(3)
    _meta_start(CPT - 1, 3)

    _gather_wait(1)
    _scale(1)
    _scatter_start(1)
    _meta_wait(CPT - 1, 3)
    _gather_start(3)
    _scatter_wait(0)

    _gather_wait(2)
    _scale(2)
    _scatter_start(2)
    _scatter_wait(1)

    _gather_wait(3)
    _scale(3)
    _scatter_start(3)
    _scatter_wait(2)
    _scatter_wait(3)

    plsc.subcore_barrier()

    def _fchunk(k, carry):
        off = (s + k * NS) * CHUNK
        pltpu.sync_copy(acc_sh.at[pl.ds(off, CHUNK)], bufs.at[0])
        pltpu.sync_copy(bufs.at[0], out_hbm.at[c, pl.ds(off, CHUNK)])
        return carry

    lax.fori_loop(0, nblk_mine, _fchunk, 0)

    @pl.when(s == 0)
    def _():
        pltpu.sync_copy(acc_sh.at[pl.ds(NRCHUNK * CHUNK, TAILR)],
                        bufs.at[0, pl.ds(0, TAILR)])
        pltpu.sync_copy(bufs.at[0, pl.ds(0, TAILR)],
                        out_hbm.at[c, pl.ds(NRCHUNK * CHUNK, TAILR)])


def kernel(input, edge_index, edge_values, W, b):
    ei = edge_index.astype(jnp.int32)
    pad = EPAD - E
    col = jnp.concatenate([ei[1], jnp.zeros((pad,), jnp.int32)])
    row = jnp.concatenate([ei[0], jnp.zeros((pad,), jnp.int32)])
    vals = jnp.concatenate([edge_values, jnp.zeros((pad,), jnp.float32)])
    vbits = lax.bitcast_convert_type(vals, jnp.int32)
    meta = jnp.stack(
        [col.reshape(NC, NS * CPT, CHUNK),
         row.reshape(NC, NS * CPT, CHUNK),
         vbits.reshape(NC, NS * CPT, CHUNK)],
        axis=2,
    )

    sup = _support(input, W.T, b.reshape(1, D))

    p = _spmm(sup, meta)
    return _reduce(p)


# speedup vs baseline: 1.4911x; 1.4911x over previous
import functools

import jax
import jax.numpy as jnp
from jax import lax
from jax.experimental import pallas as pl
from jax.experimental.pallas import tpu as pltpu
from jax.experimental.pallas import tpu_sc as plsc

N = 10000
E = 320000
D = 128

NC = 2
NS = 16
CHUNK = 128
CPT0 = 61
CPT1 = 97
EPAD = NS * (CPT0 + CPT1) * CHUNK
NRCHUNK = N // CHUNK
TAILR = N - NRCHUNK * CHUNK


def _mm_body(x_ref, wt_ref, b_ref, o_ref):
    o_ref[...] = (
        jnp.dot(x_ref[...], wt_ref[...], preferred_element_type=jnp.float32)
        + b_ref[...]
    )


def _support(x, wt, b2):
    return pl.pallas_call(
        _mm_body,
        grid=(10,),
        in_specs=[
            pl.BlockSpec((N // 10, D), lambda i: (i, 0)),
            pl.BlockSpec((D, D), lambda i: (0, 0)),
            pl.BlockSpec((1, D), lambda i: (0, 0)),
        ],
        out_specs=pl.BlockSpec((N // 10, D), lambda i: (i, 0)),
        out_shape=jax.ShapeDtypeStruct((N, D), jnp.float32),
    )(x, wt, b2)


def _add_body(p_ref, o_ref):
    o_ref[...] = p_ref[0] + p_ref[1]


def _reduce(p):
    return pl.pallas_call(
        _add_body,
        grid=(10,),
        in_specs=[pl.BlockSpec((2, N // 10, D), lambda i: (0, i, 0))],
        out_specs=pl.BlockSpec((N // 10, D), lambda i: (i, 0)),
        out_shape=jax.ShapeDtypeStruct((N, D), jnp.float32),
    )(p)


def _bcast_lane(v, i):
    idx = jnp.full((16, 1), i, jnp.int32)
    dn = lax.GatherDimensionNumbers(
        offset_dims=(), collapsed_slice_dims=(0,), start_index_map=(0,)
    )
    return lax.gather(v, idx, dn, (1,),
                      mode=lax.GatherScatterMode.PROMISE_IN_BOUNDS)


_mesh = plsc.VectorSubcoreMesh(core_axis_name="c", subcore_axis_name="s")


@functools.partial(
    pl.kernel,
    out_type=jax.ShapeDtypeStruct((2, N, D), jnp.float32),
    mesh=_mesh,
    compiler_params=pltpu.CompilerParams(use_tc_tiling_on_sc=False),
    scratch_types=[
        pltpu.VMEM((3, CHUNK), jnp.int32),
        pltpu.VMEM((CHUNK, D), jnp.float32),
        pltpu.VMEM_SHARED((N, D), jnp.float32),
        pltpu.SemaphoreType.DMA,
    ],
)
def _spmm(sup_hbm, meta_hbm, out_hbm, meta_v, buf, acc_sh, sem):
    c = lax.axis_index("c")
    s = lax.axis_index("s")
    tbase = jnp.where(c == 0, s * CPT0, NS * CPT0 + s * CPT1)
    ncht = jnp.where(c == 0, CPT0, CPT1)

    zero16 = jnp.zeros((16,), jnp.float32)

    def _zrow(r, carry):
        for f in range(D // 16):
            buf[r, pl.ds(f * 16, 16)] = zero16
        return carry

    lax.fori_loop(0, CHUNK, _zrow, 0)

    nblk_mine = NRCHUNK // NS + jnp.where(s < NRCHUNK - (NRCHUNK // NS) * NS,
                                          1, 0)

    def _zchunk(k, carry):
        off = (s + k * NS) * CHUNK
        pltpu.sync_copy(buf, acc_sh.at[pl.ds(off, CHUNK)])
        return carry

    lax.fori_loop(0, nblk_mine, _zchunk, 0)

    @pl.when(s == 0)
    def _():
        pltpu.sync_copy(buf.at[pl.ds(0, TAILR)],
                        acc_sh.at[pl.ds(NRCHUNK * CHUNK, TAILR)])

    plsc.subcore_barrier()

    def _chunk(k, carry):
        pltpu.sync_copy(meta_hbm.at[tbase + k], meta_v)
        pltpu.sync_copy(sup_hbm.at[meta_v.at[0]], buf)

        def _scale(g, inner):
            iv = meta_v[2, pl.ds(g * 16, 16)]
            vv = lax.bitcast_convert_type(iv, jnp.float32)
            for i in range(16):
                bc = _bcast_lane(vv, i)
                r = g * 16 + i
                for f in range(D // 16):
                    buf[r, pl.ds(f * 16, 16)] = (
                        buf[r, pl.ds(f * 16, 16)] * bc
                    )
            return inner

        lax.fori_loop(0, CHUNK // 16, _scale, 0)

        pltpu.async_copy(buf, acc_sh.at[meta_v.at[1]], sem, add=True)
        pltpu.make_async_copy(buf, acc_sh.at[meta_v.at[1]], sem).wait()
        return carry

    lax.fori_loop(0, ncht, _chunk, 0)

    plsc.subcore_barrier()

    def _fchunk(k, carry):
        off = (s + k * NS) * CHUNK
        pltpu.sync_copy(acc_sh.at[pl.ds(off, CHUNK)], buf)
        pltpu.sync_copy(buf, out_hbm.at[c, pl.ds(off, CHUNK)])
        return carry

    lax.fori_loop(0, nblk_mine, _fchunk, 0)

    @pl.when(s == 0)
    def _():
        pltpu.sync_copy(acc_sh.at[pl.ds(NRCHUNK * CHUNK, TAILR)],
                        buf.at[pl.ds(0, TAILR)])
        pltpu.sync_copy(buf.at[pl.ds(0, TAILR)],
                        out_hbm.at[c, pl.ds(NRCHUNK * CHUNK, TAILR)])


def kernel(input, edge_index, edge_values, W, b):
    ei = edge_index.astype(jnp.int32)
    pad = EPAD - E
    col = jnp.concatenate([ei[1], jnp.zeros((pad,), jnp.int32)])
    row = jnp.concatenate([ei[0], jnp.zeros((pad,), jnp.int32)])
    vals = jnp.concatenate([edge_values, jnp.zeros((pad,), jnp.float32)])
    vbits = lax.bitcast_convert_type(vals, jnp.int32)
    nch = NS * (CPT0 + CPT1)
    meta = jnp.stack(
        [col.reshape(nch, CHUNK),
         row.reshape(nch, CHUNK),
         vbits.reshape(nch, CHUNK)],
        axis=1,
    )

    sup = _support(input, W.T, b.reshape(1, D))

    p = _spmm(sup, meta)
    return _reduce(p)


# speedup vs baseline: 1.8610x; 1.2481x over previous
import functools

import jax
import jax.numpy as jnp
from jax import lax
from jax.experimental import pallas as pl
from jax.experimental.pallas import tpu as pltpu
from jax.experimental.pallas import tpu_sc as plsc

N = 10000
E = 320000
D = 128

NC = 2
NS = 16
CHUNK = 128
CPT0 = 97
CPT1 = 61
EPAD = NS * (CPT0 + CPT1) * CHUNK
NRCHUNK = N // CHUNK
TAILR = N - NRCHUNK * CHUNK


def _mm_body(x_ref, wt_ref, b_ref, o_ref):
    o_ref[...] = (
        jnp.dot(x_ref[...], wt_ref[...], preferred_element_type=jnp.float32)
        + b_ref[...]
    )


def _support(x, wt, b2):
    return pl.pallas_call(
        _mm_body,
        grid=(10,),
        in_specs=[
            pl.BlockSpec((N // 10, D), lambda i: (i, 0)),
            pl.BlockSpec((D, D), lambda i: (0, 0)),
            pl.BlockSpec((1, D), lambda i: (0, 0)),
        ],
        out_specs=pl.BlockSpec((N // 10, D), lambda i: (i, 0)),
        out_shape=jax.ShapeDtypeStruct((N, D), jnp.float32),
    )(x, wt, b2)


def _add_body(p_ref, o_ref):
    o_ref[...] = p_ref[0] + p_ref[1]


def _reduce(p):
    return pl.pallas_call(
        _add_body,
        grid=(10,),
        in_specs=[pl.BlockSpec((2, N // 10, D), lambda i: (0, i, 0))],
        out_specs=pl.BlockSpec((N // 10, D), lambda i: (i, 0)),
        out_shape=jax.ShapeDtypeStruct((N, D), jnp.float32),
    )(p)


def _bcast_lane(v, i):
    idx = jnp.full((16, 1), i, jnp.int32)
    dn = lax.GatherDimensionNumbers(
        offset_dims=(), collapsed_slice_dims=(0,), start_index_map=(0,)
    )
    return lax.gather(v, idx, dn, (1,),
                      mode=lax.GatherScatterMode.PROMISE_IN_BOUNDS)


_mesh = plsc.VectorSubcoreMesh(core_axis_name="c", subcore_axis_name="s")


@functools.partial(
    pl.kernel,
    out_type=jax.ShapeDtypeStruct((2, N, D), jnp.float32),
    mesh=_mesh,
    compiler_params=pltpu.CompilerParams(use_tc_tiling_on_sc=False),
    scratch_types=[
        pltpu.VMEM((3, CHUNK), jnp.int32),
        pltpu.VMEM((CHUNK, D), jnp.float32),
        pltpu.VMEM_SHARED((N, D), jnp.float32),
        pltpu.SemaphoreType.DMA,
    ],
)
def _spmm(sup_hbm, meta_hbm, out_hbm, meta_v, buf, acc_sh, sem):
    c = lax.axis_index("c")
    s = lax.axis_index("s")
    tbase = jnp.where(c == 0, s * CPT0, NS * CPT0 + s * CPT1)
    ncht = jnp.where(c == 0, CPT0, CPT1)

    zero16 = jnp.zeros((16,), jnp.float32)

    def _zrow(r, carry):
        for f in range(D // 16):
            buf[r, pl.ds(f * 16, 16)] = zero16
        return carry

    lax.fori_loop(0, CHUNK, _zrow, 0)

    nblk_mine = NRCHUNK // NS + jnp.where(s < NRCHUNK - (NRCHUNK // NS) * NS,
                                          1, 0)

    def _zchunk(k, carry):
        off = (s + k * NS) * CHUNK
        pltpu.sync_copy(buf, acc_sh.at[pl.ds(off, CHUNK)])
        return carry

    lax.fori_loop(0, nblk_mine, _zchunk, 0)

    @pl.when(s == 0)
    def _():
        pltpu.sync_copy(buf.at[pl.ds(0, TAILR)],
                        acc_sh.at[pl.ds(NRCHUNK * CHUNK, TAILR)])

    plsc.subcore_barrier()

    def _chunk(k, carry):
        pltpu.sync_copy(meta_hbm.at[tbase + k], meta_v)
        pltpu.sync_copy(sup_hbm.at[meta_v.at[0]], buf)

        def _scale(g, inner):
            iv = meta_v[2, pl.ds(g * 16, 16)]
            vv = lax.bitcast_convert_type(iv, jnp.float32)
            for i in range(16):
                bc = _bcast_lane(vv, i)
                r = g * 16 + i
                for f in range(D // 16):
                    buf[r, pl.ds(f * 16, 16)] = (
                        buf[r, pl.ds(f * 16, 16)] * bc
                    )
            return inner

        lax.fori_loop(0, CHUNK // 16, _scale, 0)

        pltpu.async_copy(buf, acc_sh.at[meta_v.at[1]], sem, add=True)
        pltpu.make_async_copy(buf, acc_sh.at[meta_v.at[1]], sem).wait()
        return carry

    lax.fori_loop(0, ncht, _chunk, 0)

    plsc.subcore_barrier()

    def _fchunk(k, carry):
        off = (s + k * NS) * CHUNK
        pltpu.sync_copy(acc_sh.at[pl.ds(off, CHUNK)], buf)
        pltpu.sync_copy(buf, out_hbm.at[c, pl.ds(off, CHUNK)])
        return carry

    lax.fori_loop(0, nblk_mine, _fchunk, 0)

    @pl.when(s == 0)
    def _():
        pltpu.sync_copy(acc_sh.at[pl.ds(NRCHUNK * CHUNK, TAILR)],
                        buf.at[pl.ds(0, TAILR)])
        pltpu.sync_copy(buf.at[pl.ds(0, TAILR)],
                        out_hbm.at[c, pl.ds(NRCHUNK * CHUNK, TAILR)])


def kernel(input, edge_index, edge_values, W, b):
    ei = edge_index.astype(jnp.int32)
    pad = EPAD - E
    col = jnp.concatenate([ei[1], jnp.zeros((pad,), jnp.int32)])
    row = jnp.concatenate([ei[0], jnp.zeros((pad,), jnp.int32)])
    vals = jnp.concatenate([edge_values, jnp.zeros((pad,), jnp.float32)])
    vbits = lax.bitcast_convert_type(vals, jnp.int32)
    nch = NS * (CPT0 + CPT1)
    meta = jnp.stack(
        [col.reshape(nch, CHUNK),
         row.reshape(nch, CHUNK),
         vbits.reshape(nch, CHUNK)],
        axis=1,
    )

    sup = _support(input, W.T, b.reshape(1, D))

    p = _spmm(sup, meta)
    return _reduce(p)
